# baseline (speedup 1.0000x reference)
import functools

import jax
import jax.numpy as jnp
from jax import lax
from jax.experimental import pallas as pl
from jax.experimental.pallas import tpu as pltpu
from jax.experimental.pallas import tpu_sc as plsc

N = 10000
NP = 10240
F_IN = 128
F = 64
E = 320000
NC = 2
NS = 16
CHUNK = 128
C = 80
JJ = 4
EPAD = NC * NS * C * CHUNK
RW = NP // NS
BR = 1024

_sc_mesh = plsc.VectorSubcoreMesh(core_axis_name="c", subcore_axis_name="s")
_sc_params = pltpu.CompilerParams(use_tc_tiling_on_sc=False)


@functools.partial(
    pl.kernel,
    out_type=jax.ShapeDtypeStruct((NC, NP), jnp.float32),
    mesh=_sc_mesh,
    scratch_types=[
        pltpu.VMEM((C // JJ, JJ * CHUNK), jnp.int32),
        pltpu.VMEM((JJ * CHUNK,), jnp.float32),
        pltpu.VMEM_SHARED((NP,), jnp.float32),
    ],
    compiler_params=_sc_params,
)
def _deg_kernel(dst_hbm, ones_hbm, zeros_hbm, out_hbm, dst_v, ones_v, deg_sh):
    c = lax.axis_index("c")
    s = lax.axis_index("s")
    pltpu.sync_copy(dst_hbm.at[c, s], dst_v)
    pltpu.sync_copy(ones_hbm, ones_v)
    pltpu.sync_copy(zeros_hbm, deg_sh.at[pl.ds(s * RW, RW)])
    plsc.subcore_barrier()

    @pl.loop(0, C // JJ)
    def _(j):
        pltpu.sync_copy(ones_v, deg_sh.at[dst_v.at[j]], add=True)

    plsc.subcore_barrier()
    pltpu.sync_copy(deg_sh.at[pl.ds(s * RW, RW)], out_hbm.at[c, pl.ds(s * RW, RW)])


@functools.partial(
    pl.kernel,
    out_type=jax.ShapeDtypeStruct((NC, NP, F), jnp.float32),
    mesh=_sc_mesh,
    scratch_types=[
        pltpu.VMEM((C // JJ, JJ * CHUNK), jnp.int32),
        pltpu.VMEM((C // JJ, JJ * CHUNK), jnp.int32),
        pltpu.VMEM((JJ * CHUNK, F), jnp.float32),
        pltpu.VMEM_SHARED((NP, F), jnp.float32),
        pltpu.SemaphoreType.DMA,
    ],
    compiler_params=_sc_params,
)
def _agg_kernel(g_hbm, src_hbm, dst_hbm, zrows_hbm, out_hbm,
                src_v, dst_v, rows_v, acc_sh, sem):
    c = lax.axis_index("c")
    s = lax.axis_index("s")
    pltpu.sync_copy(src_hbm.at[c, s], src_v)
    pltpu.sync_copy(dst_hbm.at[c, s], dst_v)
    pltpu.sync_copy(zrows_hbm, acc_sh.at[pl.ds(s * RW, RW)])
    plsc.subcore_barrier()

    @pl.loop(0, C // JJ)
    def _(j):
        pltpu.async_copy(g_hbm.at[src_v.at[j]], rows_v, sem).wait()
        pltpu.sync_copy(rows_v, acc_sh.at[dst_v.at[j]], add=True)

    plsc.subcore_barrier()
    pltpu.sync_copy(acc_sh.at[pl.ds(s * RW, RW)], out_hbm.at[c, pl.ds(s * RW, RW)])


def _tc_first_body(d0_ref, d1_ref, x_ref, w_ref, g_ref, dism_ref):
    row0 = pl.program_id(0) * BR
    rows = lax.broadcasted_iota(jnp.int32, (BR, 1), 0) + row0
    mask = (rows < N).astype(jnp.float32)
    deg = d0_ref[...] + d1_ref[...] + 1.0
    dism = jnp.broadcast_to(lax.rsqrt(deg) * mask, (BR, F))
    g_ref[...] = jnp.dot(x_ref[...], w_ref[...],
                         preferred_element_type=jnp.float32) * dism
    dism_ref[...] = dism


def _tc_first(d0, d1, x, w):
    return pl.pallas_call(
        _tc_first_body,
        grid=(NP // BR,),
        in_specs=[
            pl.BlockSpec((BR, 1), lambda i: (i, 0)),
            pl.BlockSpec((BR, 1), lambda i: (i, 0)),
            pl.BlockSpec((BR, F_IN), lambda i: (i, 0)),
            pl.BlockSpec((F_IN, F), lambda i: (0, 0)),
        ],
        out_specs=[
            pl.BlockSpec((BR, F), lambda i: (i, 0)),
            pl.BlockSpec((BR, F), lambda i: (i, 0)),
        ],
        out_shape=[
            jax.ShapeDtypeStruct((NP, F), jnp.float32),
            jax.ShapeDtypeStruct((NP, F), jnp.float32),
        ],
    )(d0, d1, x, w)


def _tc_mid_body(p0_ref, p1_ref, g_ref, dism_ref, b_ref, w_ref, gout_ref):
    agg = p0_ref[...] + p1_ref[...] + g_ref[...]
    h = jnp.maximum(agg * dism_ref[...] + b_ref[...], 0.0)
    gout_ref[...] = jnp.dot(h, w_ref[...],
                            preferred_element_type=jnp.float32) * dism_ref[...]


def _tc_mid(p0, p1, g, dism, b, w):
    return pl.pallas_call(
        _tc_mid_body,
        grid=(NP // BR,),
        in_specs=[
            pl.BlockSpec((BR, F), lambda i: (i, 0)),
            pl.BlockSpec((BR, F), lambda i: (i, 0)),
            pl.BlockSpec((BR, F), lambda i: (i, 0)),
            pl.BlockSpec((BR, F), lambda i: (i, 0)),
            pl.BlockSpec((1, F), lambda i: (0, 0)),
            pl.BlockSpec((F, F), lambda i: (0, 0)),
        ],
        out_specs=pl.BlockSpec((BR, F), lambda i: (i, 0)),
        out_shape=jax.ShapeDtypeStruct((NP, F), jnp.float32),
    )(p0, p1, g, dism, b, w)


def _tc_final_body(p0_ref, p1_ref, g_ref, dism_ref, b_ref, out_ref):
    agg = p0_ref[...] + p1_ref[...] + g_ref[...]
    out_ref[...] = agg * dism_ref[...] + b_ref[...]


def _tc_final(p0, p1, g, dism, b):
    return pl.pallas_call(
        _tc_final_body,
        grid=(NP // BR,),
        in_specs=[
            pl.BlockSpec((BR, F), lambda i: (i, 0)),
            pl.BlockSpec((BR, F), lambda i: (i, 0)),
            pl.BlockSpec((BR, F), lambda i: (i, 0)),
            pl.BlockSpec((BR, F), lambda i: (i, 0)),
            pl.BlockSpec((1, F), lambda i: (0, 0)),
        ],
        out_specs=pl.BlockSpec((BR, F), lambda i: (i, 0)),
        out_shape=jax.ShapeDtypeStruct((NP, F), jnp.float32),
    )(p0, p1, g, dism, b)


def kernel(x, edge_index, W0, b0, W1, b1, W2, b2):
    src = edge_index[0].astype(jnp.int32)
    dst = edge_index[1].astype(jnp.int32)
    pad = jnp.full((EPAD - E,), N, dtype=jnp.int32)
    src3 = jnp.concatenate([src, pad]).reshape(NC, NS, C // JJ, JJ * CHUNK)
    dst3 = jnp.concatenate([dst, pad]).reshape(NC, NS, C // JJ, JJ * CHUNK)

    ones_chunk = jnp.ones((JJ * CHUNK,), jnp.float32)
    zeros_deg = jnp.zeros((RW,), jnp.float32)
    zeros_rows = jnp.zeros((RW, F), jnp.float32)
    x_pad = jnp.concatenate([x, jnp.zeros((NP - N, F_IN), x.dtype)])

    degp = _deg_kernel(dst3, ones_chunk, zeros_deg)
    d0 = degp[0].reshape(NP, 1)
    d1 = degp[1].reshape(NP, 1)

    b0r = b0.reshape(1, F)
    b1r = b1.reshape(1, F)
    b2r = b2.reshape(1, F)

    g0, dism = _tc_first(d0, d1, x_pad, W0)
    p = _agg_kernel(g0, src3, dst3, zeros_rows)
    g1 = _tc_mid(p[0], p[1], g0, dism, b0r, W1)
    p = _agg_kernel(g1, src3, dst3, zeros_rows)
    g2 = _tc_mid(p[0], p[1], g1, dism, b1r, W2)
    p = _agg_kernel(g2, src3, dst3, zeros_rows)
    out = _tc_final(p[0], p[1], g2, dism, b2r)
    return out[:N]


# speedup vs baseline: 1.8126x; 1.8126x over previous
import functools

import jax
import jax.numpy as jnp
from jax import lax
from jax.experimental import pallas as pl
from jax.experimental.pallas import tpu as pltpu
from jax.experimental.pallas import tpu_sc as plsc

N = 10000
NP = 10240
F_IN = 128
F = 64
E = 320000
NC = 2
NS = 16
CHUNK = 128
C = 80
JJ = 1
EPAD = NC * NS * C * CHUNK
RW = NP // NS
BR = 1024

_sc_mesh = plsc.VectorSubcoreMesh(core_axis_name="c", subcore_axis_name="s")
_sc_params = pltpu.CompilerParams(use_tc_tiling_on_sc=False)


@functools.partial(
    pl.kernel,
    out_type=jax.ShapeDtypeStruct((NC, NP), jnp.float32),
    mesh=_sc_mesh,
    scratch_types=[
        pltpu.VMEM((C // JJ, JJ * CHUNK), jnp.int32),
        pltpu.VMEM((JJ * CHUNK,), jnp.float32),
        pltpu.VMEM_SHARED((NP,), jnp.float32),
    ],
    compiler_params=_sc_params,
)
def _deg_kernel(dst_hbm, ones_hbm, zeros_hbm, out_hbm, dst_v, ones_v, deg_sh):
    c = lax.axis_index("c")
    s = lax.axis_index("s")
    pltpu.sync_copy(dst_hbm.at[c, s], dst_v)
    pltpu.sync_copy(ones_hbm, ones_v)
    pltpu.sync_copy(zeros_hbm, deg_sh.at[pl.ds(s * RW, RW)])
    plsc.subcore_barrier()

    @pl.loop(0, C // JJ)
    def _(j):
        pltpu.sync_copy(ones_v, deg_sh.at[dst_v.at[j]], add=True)

    plsc.subcore_barrier()
    pltpu.sync_copy(deg_sh.at[pl.ds(s * RW, RW)], out_hbm.at[c, pl.ds(s * RW, RW)])


@functools.partial(
    pl.kernel,
    out_type=jax.ShapeDtypeStruct((NC, NP, F), jnp.float32),
    mesh=_sc_mesh,
    scratch_types=[
        pltpu.VMEM((C // JJ, JJ * CHUNK), jnp.int32),
        pltpu.VMEM((C // JJ, JJ * CHUNK), jnp.int32),
        pltpu.VMEM((JJ * CHUNK, F), jnp.float32),
        pltpu.VMEM_SHARED((NP, F), jnp.float32),
        pltpu.VMEM_SHARED((NP, F), jnp.float32),
        pltpu.SemaphoreType.DMA,
    ],
    compiler_params=_sc_params,
)
def _agg_kernel(g_hbm, src_hbm, dst_hbm, zrows_hbm, out_hbm,
                src_v, dst_v, rows_v, acc_sh, g_sh, sem):
    c = lax.axis_index("c")
    s = lax.axis_index("s")
    pltpu.sync_copy(src_hbm.at[c, s], src_v)
    pltpu.sync_copy(dst_hbm.at[c, s], dst_v)
    pltpu.sync_copy(zrows_hbm, acc_sh.at[pl.ds(s * RW, RW)])
    pltpu.sync_copy(g_hbm.at[pl.ds(s * RW, RW)], g_sh.at[pl.ds(s * RW, RW)])
    plsc.subcore_barrier()

    @pl.loop(0, C // JJ)
    def _(j):
        pltpu.async_copy(g_sh.at[src_v.at[j]], rows_v, sem).wait()
        pltpu.sync_copy(rows_v, acc_sh.at[dst_v.at[j]], add=True)

    plsc.subcore_barrier()
    pltpu.sync_copy(acc_sh.at[pl.ds(s * RW, RW)], out_hbm.at[c, pl.ds(s * RW, RW)])


def _tc_first_body(d0_ref, d1_ref, x_ref, w_ref, g_ref, dism_ref):
    row0 = pl.program_id(0) * BR
    rows = lax.broadcasted_iota(jnp.int32, (BR, 1), 0) + row0
    mask = (rows < N).astype(jnp.float32)
    deg = d0_ref[...] + d1_ref[...] + 1.0
    dism = jnp.broadcast_to(lax.rsqrt(deg) * mask, (BR, F))
    g_ref[...] = jnp.dot(x_ref[...], w_ref[...],
                         preferred_element_type=jnp.float32) * dism
    dism_ref[...] = dism


def _tc_first(d0, d1, x, w):
    return pl.pallas_call(
        _tc_first_body,
        grid=(NP // BR,),
        in_specs=[
            pl.BlockSpec((BR, 1), lambda i: (i, 0)),
            pl.BlockSpec((BR, 1), lambda i: (i, 0)),
            pl.BlockSpec((BR, F_IN), lambda i: (i, 0)),
            pl.BlockSpec((F_IN, F), lambda i: (0, 0)),
        ],
        out_specs=[
            pl.BlockSpec((BR, F), lambda i: (i, 0)),
            pl.BlockSpec((BR, F), lambda i: (i, 0)),
        ],
        out_shape=[
            jax.ShapeDtypeStruct((NP, F), jnp.float32),
            jax.ShapeDtypeStruct((NP, F), jnp.float32),
        ],
    )(d0, d1, x, w)


def _tc_mid_body(p0_ref, p1_ref, g_ref, dism_ref, b_ref, w_ref, gout_ref):
    agg = p0_ref[...] + p1_ref[...] + g_ref[...]
    h = jnp.maximum(agg * dism_ref[...] + b_ref[...], 0.0)
    gout_ref[...] = jnp.dot(h, w_ref[...],
                            preferred_element_type=jnp.float32) * dism_ref[...]


def _tc_mid(p0, p1, g, dism, b, w):
    return pl.pallas_call(
        _tc_mid_body,
        grid=(NP // BR,),
        in_specs=[
            pl.BlockSpec((BR, F), lambda i: (i, 0)),
            pl.BlockSpec((BR, F), lambda i: (i, 0)),
            pl.BlockSpec((BR, F), lambda i: (i, 0)),
            pl.BlockSpec((BR, F), lambda i: (i, 0)),
            pl.BlockSpec((1, F), lambda i: (0, 0)),
            pl.BlockSpec((F, F), lambda i: (0, 0)),
        ],
        out_specs=pl.BlockSpec((BR, F), lambda i: (i, 0)),
        out_shape=jax.ShapeDtypeStruct((NP, F), jnp.float32),
    )(p0, p1, g, dism, b, w)


def _tc_final_body(p0_ref, p1_ref, g_ref, dism_ref, b_ref, out_ref):
    agg = p0_ref[...] + p1_ref[...] + g_ref[...]
    out_ref[...] = agg * dism_ref[...] + b_ref[...]


def _tc_final(p0, p1, g, dism, b):
    return pl.pallas_call(
        _tc_final_body,
        grid=(NP // BR,),
        in_specs=[
            pl.BlockSpec((BR, F), lambda i: (i, 0)),
            pl.BlockSpec((BR, F), lambda i: (i, 0)),
            pl.BlockSpec((BR, F), lambda i: (i, 0)),
            pl.BlockSpec((BR, F), lambda i: (i, 0)),
            pl.BlockSpec((1, F), lambda i: (0, 0)),
        ],
        out_specs=pl.BlockSpec((BR, F), lambda i: (i, 0)),
        out_shape=jax.ShapeDtypeStruct((NP, F), jnp.float32),
    )(p0, p1, g, dism, b)


def kernel(x, edge_index, W0, b0, W1, b1, W2, b2):
    src = edge_index[0].astype(jnp.int32)
    dst = edge_index[1].astype(jnp.int32)
    pad = jnp.full((EPAD - E,), N, dtype=jnp.int32)
    src3 = jnp.concatenate([src, pad]).reshape(NC, NS, C // JJ, JJ * CHUNK)
    dst3 = jnp.concatenate([dst, pad]).reshape(NC, NS, C // JJ, JJ * CHUNK)

    ones_chunk = jnp.ones((JJ * CHUNK,), jnp.float32)
    zeros_deg = jnp.zeros((RW,), jnp.float32)
    zeros_rows = jnp.zeros((RW, F), jnp.float32)
    x_pad = jnp.concatenate([x, jnp.zeros((NP - N, F_IN), x.dtype)])

    degp = _deg_kernel(dst3, ones_chunk, zeros_deg)
    d0 = degp[0].reshape(NP, 1)
    d1 = degp[1].reshape(NP, 1)

    b0r = b0.reshape(1, F)
    b1r = b1.reshape(1, F)
    b2r = b2.reshape(1, F)

    g0, dism = _tc_first(d0, d1, x_pad, W0)
    p = _agg_kernel(g0, src3, dst3, zeros_rows)
    g1 = _tc_mid(p[0], p[1], g0, dism, b0r, W1)
    p = _agg_kernel(g1, src3, dst3, zeros_rows)
    g2 = _tc_mid(p[0], p[1], g1, dism, b1r, W2)
    p = _agg_kernel(g2, src3, dst3, zeros_rows)
    out = _tc_final(p[0], p[1], g2, dism, b2r)
    return out[:N]


# speedup vs baseline: 2.2434x; 1.2377x over previous
import functools

import jax
import jax.numpy as jnp
from jax import lax
from jax.experimental import pallas as pl
from jax.experimental.pallas import tpu as pltpu
from jax.experimental.pallas import tpu_sc as plsc

N = 10000
NP = 10240
F_IN = 128
F = 64
E = 320000
NC = 2
NS = 16
CHUNK = 128
C = 79
JJ = 1
EPAD = NC * NS * C * CHUNK
RW = NP // NS
BR = 1024

_sc_mesh = plsc.VectorSubcoreMesh(core_axis_name="c", subcore_axis_name="s")
_sc_params = pltpu.CompilerParams(use_tc_tiling_on_sc=False)


@functools.partial(
    pl.kernel,
    out_type=jax.ShapeDtypeStruct((NC, NP), jnp.float32),
    mesh=_sc_mesh,
    scratch_types=[
        pltpu.VMEM((C // JJ, JJ * CHUNK), jnp.int32),
        pltpu.VMEM((JJ * CHUNK,), jnp.float32),
        pltpu.VMEM_SHARED((NP,), jnp.float32),
    ],
    compiler_params=_sc_params,
)
def _deg_kernel(dst_hbm, ones_hbm, zeros_hbm, out_hbm, dst_v, ones_v, deg_sh):
    c = lax.axis_index("c")
    s = lax.axis_index("s")
    pltpu.sync_copy(dst_hbm.at[c, s], dst_v)
    pltpu.sync_copy(ones_hbm, ones_v)
    pltpu.sync_copy(zeros_hbm, deg_sh.at[pl.ds(s * RW, RW)])
    plsc.subcore_barrier()

    @pl.loop(0, C // JJ)
    def _(j):
        pltpu.sync_copy(ones_v, deg_sh.at[dst_v.at[j]], add=True)

    plsc.subcore_barrier()
    pltpu.sync_copy(deg_sh.at[pl.ds(s * RW, RW)], out_hbm.at[c, pl.ds(s * RW, RW)])


@functools.partial(
    pl.kernel,
    out_type=jax.ShapeDtypeStruct((NC, NP, F), jnp.float32),
    mesh=_sc_mesh,
    scratch_types=[
        pltpu.VMEM((C // JJ, JJ * CHUNK), jnp.int32),
        pltpu.VMEM((C // JJ, JJ * CHUNK), jnp.int32),
        pltpu.VMEM((2, JJ * CHUNK, F), jnp.float32),
        pltpu.VMEM_SHARED((NP, F), jnp.float32),
        pltpu.VMEM_SHARED((NP, F), jnp.float32),
        pltpu.SemaphoreType.DMA,
        pltpu.SemaphoreType.DMA,
    ],
    compiler_params=_sc_params,
)
def _agg_kernel(g_hbm, src_hbm, dst_hbm, zrows_hbm, out_hbm,
                src_v, dst_v, rows_v, acc_sh, g_sh, sem0, sem1):
    c = lax.axis_index("c")
    s = lax.axis_index("s")
    pltpu.sync_copy(src_hbm.at[c, s], src_v)
    pltpu.sync_copy(dst_hbm.at[c, s], dst_v)
    pltpu.sync_copy(zrows_hbm, acc_sh.at[pl.ds(s * RW, RW)])
    pltpu.sync_copy(g_hbm.at[pl.ds(s * RW, RW)], g_sh.at[pl.ds(s * RW, RW)])
    plsc.subcore_barrier()

    pltpu.async_copy(g_sh.at[src_v.at[0]], rows_v.at[0], sem0)

    @pl.loop(0, C - 1, step=2)
    def _(j):
        pltpu.make_async_copy(g_sh.at[src_v.at[j]], rows_v.at[0], sem0).wait()
        pltpu.async_copy(g_sh.at[src_v.at[j + 1]], rows_v.at[1], sem1)
        pltpu.sync_copy(rows_v.at[0], acc_sh.at[dst_v.at[j]], add=True)
        pltpu.make_async_copy(g_sh.at[src_v.at[j + 1]], rows_v.at[1],
                              sem1).wait()
        pltpu.async_copy(g_sh.at[src_v.at[j + 2]], rows_v.at[0], sem0)
        pltpu.sync_copy(rows_v.at[1], acc_sh.at[dst_v.at[j + 1]], add=True)

    pltpu.make_async_copy(g_sh.at[src_v.at[C - 1]], rows_v.at[0], sem0).wait()
    pltpu.sync_copy(rows_v.at[0], acc_sh.at[dst_v.at[C - 1]], add=True)

    plsc.subcore_barrier()
    pltpu.sync_copy(acc_sh.at[pl.ds(s * RW, RW)], out_hbm.at[c, pl.ds(s * RW, RW)])


def _tc_first_body(d0_ref, d1_ref, x_ref, w_ref, g_ref, dism_ref):
    row0 = pl.program_id(0) * BR
    rows = lax.broadcasted_iota(jnp.int32, (BR, 1), 0) + row0
    mask = (rows < N).astype(jnp.float32)
    deg = d0_ref[...] + d1_ref[...] + 1.0
    dism = jnp.broadcast_to(lax.rsqrt(deg) * mask, (BR, F))
    g_ref[...] = jnp.dot(x_ref[...], w_ref[...],
                         preferred_element_type=jnp.float32) * dism
    dism_ref[...] = dism


def _tc_first(d0, d1, x, w):
    return pl.pallas_call(
        _tc_first_body,
        grid=(NP // BR,),
        in_specs=[
            pl.BlockSpec((BR, 1), lambda i: (i, 0)),
            pl.BlockSpec((BR, 1), lambda i: (i, 0)),
            pl.BlockSpec((BR, F_IN), lambda i: (i, 0)),
            pl.BlockSpec((F_IN, F), lambda i: (0, 0)),
        ],
        out_specs=[
            pl.BlockSpec((BR, F), lambda i: (i, 0)),
            pl.BlockSpec((BR, F), lambda i: (i, 0)),
        ],
        out_shape=[
            jax.ShapeDtypeStruct((NP, F), jnp.float32),
            jax.ShapeDtypeStruct((NP, F), jnp.float32),
        ],
    )(d0, d1, x, w)


def _tc_mid_body(p0_ref, p1_ref, g_ref, dism_ref, b_ref, w_ref, gout_ref):
    agg = p0_ref[...] + p1_ref[...] + g_ref[...]
    h = jnp.maximum(agg * dism_ref[...] + b_ref[...], 0.0)
    gout_ref[...] = jnp.dot(h, w_ref[...],
                            preferred_element_type=jnp.float32) * dism_ref[...]


def _tc_mid(p0, p1, g, dism, b, w):
    return pl.pallas_call(
        _tc_mid_body,
        grid=(NP // BR,),
        in_specs=[
            pl.BlockSpec((BR, F), lambda i: (i, 0)),
            pl.BlockSpec((BR, F), lambda i: (i, 0)),
            pl.BlockSpec((BR, F), lambda i: (i, 0)),
            pl.BlockSpec((BR, F), lambda i: (i, 0)),
            pl.BlockSpec((1, F), lambda i: (0, 0)),
            pl.BlockSpec((F, F), lambda i: (0, 0)),
        ],
        out_specs=pl.BlockSpec((BR, F), lambda i: (i, 0)),
        out_shape=jax.ShapeDtypeStruct((NP, F), jnp.float32),
    )(p0, p1, g, dism, b, w)


def _tc_final_body(p0_ref, p1_ref, g_ref, dism_ref, b_ref, out_ref):
    agg = p0_ref[...] + p1_ref[...] + g_ref[...]
    out_ref[...] = agg * dism_ref[...] + b_ref[...]


def _tc_final(p0, p1, g, dism, b):
    return pl.pallas_call(
        _tc_final_body,
        grid=(NP // BR,),
        in_specs=[
            pl.BlockSpec((BR, F), lambda i: (i, 0)),
            pl.BlockSpec((BR, F), lambda i: (i, 0)),
            pl.BlockSpec((BR, F), lambda i: (i, 0)),
            pl.BlockSpec((BR, F), lambda i: (i, 0)),
            pl.BlockSpec((1, F), lambda i: (0, 0)),
        ],
        out_specs=pl.BlockSpec((BR, F), lambda i: (i, 0)),
        out_shape=jax.ShapeDtypeStruct((NP, F), jnp.float32),
    )(p0, p1, g, dism, b)


def kernel(x, edge_index, W0, b0, W1, b1, W2, b2):
    src = edge_index[0].astype(jnp.int32)
    dst = edge_index[1].astype(jnp.int32)
    pad = jnp.full((EPAD - E,), N, dtype=jnp.int32)
    src3 = jnp.concatenate([src, pad]).reshape(NC, NS, C // JJ, JJ * CHUNK)
    dst3 = jnp.concatenate([dst, pad]).reshape(NC, NS, C // JJ, JJ * CHUNK)

    ones_chunk = jnp.ones((JJ * CHUNK,), jnp.float32)
    zeros_deg = jnp.zeros((RW,), jnp.float32)
    zeros_rows = jnp.zeros((RW, F), jnp.float32)
    x_pad = jnp.concatenate([x, jnp.zeros((NP - N, F_IN), x.dtype)])

    degp = _deg_kernel(dst3, ones_chunk, zeros_deg)
    d0 = degp[0].reshape(NP, 1)
    d1 = degp[1].reshape(NP, 1)

    b0r = b0.reshape(1, F)
    b1r = b1.reshape(1, F)
    b2r = b2.reshape(1, F)

    g0, dism = _tc_first(d0, d1, x_pad, W0)
    p = _agg_kernel(g0, src3, dst3, zeros_rows)
    g1 = _tc_mid(p[0], p[1], g0, dism, b0r, W1)
    p = _agg_kernel(g1, src3, dst3, zeros_rows)
    g2 = _tc_mid(p[0], p[1], g1, dism, b1r, W2)
    p = _agg_kernel(g2, src3, dst3, zeros_rows)
    out = _tc_final(p[0], p[1], g2, dism, b2r)
    return out[:N]


# speedup vs baseline: 2.2609x; 1.0078x over previous
import functools

import jax
import jax.numpy as jnp
from jax import lax
from jax.experimental import pallas as pl
from jax.experimental.pallas import tpu as pltpu
from jax.experimental.pallas import tpu_sc as plsc

N = 10000
NP = 10240
F_IN = 128
F = 64
E = 320000
NC = 2
NS = 16
CHUNK = 128
C = 79
JJ = 1
EPAD = NC * NS * C * CHUNK
RW = NP // NS
BR = 1024

_sc_mesh = plsc.VectorSubcoreMesh(core_axis_name="c", subcore_axis_name="s")
_sc_params = pltpu.CompilerParams(use_tc_tiling_on_sc=False)


@functools.partial(
    pl.kernel,
    out_type=jax.ShapeDtypeStruct((NC, NP), jnp.float32),
    mesh=_sc_mesh,
    scratch_types=[
        pltpu.VMEM((C // JJ, JJ * CHUNK), jnp.int32),
        pltpu.VMEM((JJ * CHUNK,), jnp.float32),
        pltpu.VMEM_SHARED((NP,), jnp.float32),
    ],
    compiler_params=_sc_params,
)
def _deg_kernel(dst_hbm, ones_hbm, zeros_hbm, out_hbm, dst_v, ones_v, deg_sh):
    c = lax.axis_index("c")
    s = lax.axis_index("s")
    pltpu.sync_copy(dst_hbm.at[c, s], dst_v)
    pltpu.sync_copy(ones_hbm, ones_v)
    pltpu.sync_copy(zeros_hbm, deg_sh.at[pl.ds(s * RW, RW)])
    plsc.subcore_barrier()

    @pl.loop(0, C // JJ)
    def _(j):
        pltpu.sync_copy(ones_v, deg_sh.at[dst_v.at[j]], add=True)

    plsc.subcore_barrier()
    pltpu.sync_copy(deg_sh.at[pl.ds(s * RW, RW)], out_hbm.at[c, pl.ds(s * RW, RW)])


@functools.partial(
    pl.kernel,
    out_type=jax.ShapeDtypeStruct((NC, NP, F), jnp.float32),
    mesh=_sc_mesh,
    scratch_types=[
        pltpu.VMEM((C // JJ, JJ * CHUNK), jnp.int32),
        pltpu.VMEM((C // JJ, JJ * CHUNK), jnp.int32),
        pltpu.VMEM((2, JJ * CHUNK, F), jnp.float32),
        pltpu.VMEM_SHARED((NP, F), jnp.float32),
        pltpu.VMEM_SHARED((NP, F), jnp.float32),
        pltpu.SemaphoreType.DMA,
        pltpu.SemaphoreType.DMA,
        pltpu.SemaphoreType.DMA,
        pltpu.SemaphoreType.DMA,
    ],
    compiler_params=_sc_params,
)
def _agg_kernel(g_hbm, src_hbm, dst_hbm, zrows_hbm, out_hbm,
                src_v, dst_v, rows_v, acc_sh, g_sh, sem0, sem1, sem2, sem3):
    c = lax.axis_index("c")
    s = lax.axis_index("s")
    pltpu.async_copy(src_hbm.at[c, s], src_v, sem0)
    pltpu.async_copy(dst_hbm.at[c, s], dst_v, sem1)
    pltpu.async_copy(zrows_hbm, acc_sh.at[pl.ds(s * RW, RW)], sem2)
    pltpu.async_copy(g_hbm.at[pl.ds(s * RW, RW)], g_sh.at[pl.ds(s * RW, RW)],
                     sem3)
    pltpu.make_async_copy(src_hbm.at[c, s], src_v, sem0).wait()
    pltpu.make_async_copy(dst_hbm.at[c, s], dst_v, sem1).wait()
    pltpu.make_async_copy(zrows_hbm, acc_sh.at[pl.ds(s * RW, RW)], sem2).wait()
    pltpu.make_async_copy(g_hbm.at[pl.ds(s * RW, RW)],
                          g_sh.at[pl.ds(s * RW, RW)], sem3).wait()
    plsc.subcore_barrier()

    pltpu.async_copy(g_sh.at[src_v.at[0]], rows_v.at[0], sem0)

    @pl.loop(0, C - 1, step=2)
    def _(j):
        pltpu.make_async_copy(g_sh.at[src_v.at[j]], rows_v.at[0], sem0).wait()
        pltpu.async_copy(g_sh.at[src_v.at[j + 1]], rows_v.at[1], sem1)
        pltpu.sync_copy(rows_v.at[0], acc_sh.at[dst_v.at[j]], add=True)
        pltpu.make_async_copy(g_sh.at[src_v.at[j + 1]], rows_v.at[1],
                              sem1).wait()
        pltpu.async_copy(g_sh.at[src_v.at[j + 2]], rows_v.at[0], sem0)
        pltpu.sync_copy(rows_v.at[1], acc_sh.at[dst_v.at[j + 1]], add=True)

    pltpu.make_async_copy(g_sh.at[src_v.at[C - 1]], rows_v.at[0], sem0).wait()
    pltpu.sync_copy(rows_v.at[0], acc_sh.at[dst_v.at[C - 1]], add=True)

    plsc.subcore_barrier()
    pltpu.sync_copy(acc_sh.at[pl.ds(s * RW, RW)], out_hbm.at[c, pl.ds(s * RW, RW)])


def _tc_first_body(d0_ref, d1_ref, x_ref, w_ref, g_ref, dism_ref):
    row0 = pl.program_id(0) * BR
    rows = lax.broadcasted_iota(jnp.int32, (BR, 1), 0) + row0
    mask = (rows < N).astype(jnp.float32)
    deg = d0_ref[...] + d1_ref[...] + 1.0
    dism = lax.rsqrt(deg) * mask
    g_ref[...] = jnp.dot(x_ref[...], w_ref[...],
                         preferred_element_type=jnp.float32) * dism
    dism_ref[...] = dism


def _tc_first(d0, d1, x, w):
    return pl.pallas_call(
        _tc_first_body,
        grid=(NP // BR,),
        in_specs=[
            pl.BlockSpec((BR, 1), lambda i: (i, 0)),
            pl.BlockSpec((BR, 1), lambda i: (i, 0)),
            pl.BlockSpec((BR, F_IN), lambda i: (i, 0)),
            pl.BlockSpec((F_IN, F), lambda i: (0, 0)),
        ],
        out_specs=[
            pl.BlockSpec((BR, F), lambda i: (i, 0)),
            pl.BlockSpec((BR, 1), lambda i: (i, 0)),
        ],
        out_shape=[
            jax.ShapeDtypeStruct((NP, F), jnp.float32),
            jax.ShapeDtypeStruct((NP, 1), jnp.float32),
        ],
    )(d0, d1, x, w)


def _tc_mid_body(p0_ref, p1_ref, g_ref, dism_ref, b_ref, w_ref, gout_ref):
    dism = dism_ref[...]
    agg = p0_ref[...] + p1_ref[...] + g_ref[...]
    h = jnp.maximum(agg * dism + b_ref[...], 0.0)
    gout_ref[...] = jnp.dot(h, w_ref[...],
                            preferred_element_type=jnp.float32) * dism


def _tc_mid(p0, p1, g, dism, b, w):
    return pl.pallas_call(
        _tc_mid_body,
        grid=(NP // BR,),
        in_specs=[
            pl.BlockSpec((BR, F), lambda i: (i, 0)),
            pl.BlockSpec((BR, F), lambda i: (i, 0)),
            pl.BlockSpec((BR, F), lambda i: (i, 0)),
            pl.BlockSpec((BR, 1), lambda i: (i, 0)),
            pl.BlockSpec((1, F), lambda i: (0, 0)),
            pl.BlockSpec((F, F), lambda i: (0, 0)),
        ],
        out_specs=pl.BlockSpec((BR, F), lambda i: (i, 0)),
        out_shape=jax.ShapeDtypeStruct((NP, F), jnp.float32),
    )(p0, p1, g, dism, b, w)


def _tc_final_body(p0_ref, p1_ref, g_ref, dism_ref, b_ref, out_ref):
    agg = p0_ref[...] + p1_ref[...] + g_ref[...]
    out_ref[...] = agg * dism_ref[...] + b_ref[...]


def _tc_final(p0, p1, g, dism, b):
    return pl.pallas_call(
        _tc_final_body,
        grid=(NP // BR,),
        in_specs=[
            pl.BlockSpec((BR, F), lambda i: (i, 0)),
            pl.BlockSpec((BR, F), lambda i: (i, 0)),
            pl.BlockSpec((BR, F), lambda i: (i, 0)),
            pl.BlockSpec((BR, 1), lambda i: (i, 0)),
            pl.BlockSpec((1, F), lambda i: (0, 0)),
        ],
        out_specs=pl.BlockSpec((BR, F), lambda i: (i, 0)),
        out_shape=jax.ShapeDtypeStruct((NP, F), jnp.float32),
    )(p0, p1, g, dism, b)


def kernel(x, edge_index, W0, b0, W1, b1, W2, b2):
    src = edge_index[0].astype(jnp.int32)
    dst = edge_index[1].astype(jnp.int32)
    pad = jnp.full((EPAD - E,), N, dtype=jnp.int32)
    src3 = jnp.concatenate([src, pad]).reshape(NC, NS, C // JJ, JJ * CHUNK)
    dst3 = jnp.concatenate([dst, pad]).reshape(NC, NS, C // JJ, JJ * CHUNK)

    ones_chunk = jnp.ones((JJ * CHUNK,), jnp.float32)
    zeros_deg = jnp.zeros((RW,), jnp.float32)
    zeros_rows = jnp.zeros((RW, F), jnp.float32)
    x_pad = jnp.concatenate([x, jnp.zeros((NP - N, F_IN), x.dtype)])

    degp = _deg_kernel(dst3, ones_chunk, zeros_deg)
    d0 = degp[0].reshape(NP, 1)
    d1 = degp[1].reshape(NP, 1)

    b0r = b0.reshape(1, F)
    b1r = b1.reshape(1, F)
    b2r = b2.reshape(1, F)

    g0, dism = _tc_first(d0, d1, x_pad, W0)
    p = _agg_kernel(g0, src3, dst3, zeros_rows)
    g1 = _tc_mid(p[0], p[1], g0, dism, b0r, W1)
    p = _agg_kernel(g1, src3, dst3, zeros_rows)
    g2 = _tc_mid(p[0], p[1], g1, dism, b1r, W2)
    p = _agg_kernel(g2, src3, dst3, zeros_rows)
    out = _tc_final(p[0], p[1], g2, dism, b2r)
    return out[:N]


# speedup vs baseline: 2.3361x; 1.0333x over previous
import functools

import jax
import jax.numpy as jnp
from jax import lax
from jax.experimental import pallas as pl
from jax.experimental.pallas import tpu as pltpu
from jax.experimental.pallas import tpu_sc as plsc

N = 10000
NP = 10240
F_IN = 128
F = 64
E = 320000
NC = 2
NS = 16
CHUNK = 128
C = 79
EPAD = NC * NS * C * CHUNK
RW = NP // NS
RWO = N // NS
BR = 2000

_sc_mesh = plsc.VectorSubcoreMesh(core_axis_name="c", subcore_axis_name="s")
_sc_params = pltpu.CompilerParams(use_tc_tiling_on_sc=False)


@functools.partial(
    pl.kernel,
    out_type=jax.ShapeDtypeStruct((NC, NP), jnp.float32),
    mesh=_sc_mesh,
    scratch_types=[
        pltpu.VMEM((C, CHUNK), jnp.int32),
        pltpu.VMEM((CHUNK,), jnp.float32),
        pltpu.VMEM_SHARED((NP,), jnp.float32),
    ],
    compiler_params=_sc_params,
)
def _deg_kernel(dst_hbm, ones_hbm, zeros_hbm, out_hbm, dst_v, ones_v, deg_sh):
    c = lax.axis_index("c")
    s = lax.axis_index("s")
    pltpu.sync_copy(dst_hbm.at[c, s], dst_v)
    pltpu.sync_copy(ones_hbm, ones_v)
    pltpu.sync_copy(zeros_hbm, deg_sh.at[pl.ds(s * RW, RW)])
    plsc.subcore_barrier()

    @pl.loop(0, C)
    def _(j):
        pltpu.sync_copy(ones_v, deg_sh.at[dst_v.at[j]], add=True)

    plsc.subcore_barrier()
    pltpu.sync_copy(deg_sh.at[pl.ds(s * RW, RW)], out_hbm.at[c, pl.ds(s * RW, RW)])


@functools.partial(
    pl.kernel,
    out_type=jax.ShapeDtypeStruct((NC, N, F), jnp.float32),
    mesh=_sc_mesh,
    scratch_types=[
        pltpu.VMEM((C, CHUNK), jnp.int32),
        pltpu.VMEM((C, CHUNK), jnp.int32),
        pltpu.VMEM((2, CHUNK, F), jnp.float32),
        pltpu.VMEM_SHARED((NP, F), jnp.float32),
        pltpu.VMEM_SHARED((N, F), jnp.float32),
        pltpu.SemaphoreType.DMA,
        pltpu.SemaphoreType.DMA,
        pltpu.SemaphoreType.DMA,
        pltpu.SemaphoreType.DMA,
    ],
    compiler_params=_sc_params,
)
def _agg_kernel(g_hbm, src_hbm, dst_hbm, zrows_hbm, out_hbm,
                src_v, dst_v, rows_v, acc_sh, g_sh, sem0, sem1, sem2, sem3):
    c = lax.axis_index("c")
    s = lax.axis_index("s")
    pltpu.async_copy(src_hbm.at[c, s], src_v, sem0)
    pltpu.async_copy(dst_hbm.at[c, s], dst_v, sem1)
    pltpu.async_copy(zrows_hbm, acc_sh.at[pl.ds(s * RWO, RWO)], sem2)
    pltpu.async_copy(g_hbm.at[pl.ds(s * RWO, RWO)],
                     g_sh.at[pl.ds(s * RWO, RWO)], sem3)
    pltpu.make_async_copy(src_hbm.at[c, s], src_v, sem0).wait()
    pltpu.make_async_copy(dst_hbm.at[c, s], dst_v, sem1).wait()
    pltpu.make_async_copy(zrows_hbm, acc_sh.at[pl.ds(s * RWO, RWO)],
                          sem2).wait()
    pltpu.make_async_copy(g_hbm.at[pl.ds(s * RWO, RWO)],
                          g_sh.at[pl.ds(s * RWO, RWO)], sem3).wait()
    plsc.subcore_barrier()

    pltpu.async_copy(g_sh.at[src_v.at[0]], rows_v.at[0], sem0)

    @pl.loop(0, C - 1, step=2)
    def _(j):
        pltpu.make_async_copy(g_sh.at[src_v.at[j]], rows_v.at[0], sem0).wait()
        pltpu.async_copy(g_sh.at[src_v.at[j + 1]], rows_v.at[1], sem1)
        pltpu.sync_copy(rows_v.at[0], acc_sh.at[dst_v.at[j]], add=True)
        pltpu.make_async_copy(g_sh.at[src_v.at[j + 1]], rows_v.at[1],
                              sem1).wait()
        pltpu.async_copy(g_sh.at[src_v.at[j + 2]], rows_v.at[0], sem0)
        pltpu.sync_copy(rows_v.at[1], acc_sh.at[dst_v.at[j + 1]], add=True)

    pltpu.make_async_copy(g_sh.at[src_v.at[C - 1]], rows_v.at[0], sem0).wait()
    pltpu.sync_copy(rows_v.at[0], acc_sh.at[dst_v.at[C - 1]], add=True)

    plsc.subcore_barrier()
    pltpu.sync_copy(acc_sh.at[pl.ds(s * RWO, RWO)],
                    out_hbm.at[c, pl.ds(s * RWO, RWO)])


def _tc_first_body(d0_ref, d1_ref, x_ref, w_ref, g_ref, dism_ref):
    dism = lax.rsqrt(d0_ref[...] + d1_ref[...] + 1.0)
    g_ref[...] = jnp.dot(x_ref[...], w_ref[...],
                         preferred_element_type=jnp.float32) * dism
    dism_ref[...] = dism


def _tc_first(d0, d1, x, w):
    return pl.pallas_call(
        _tc_first_body,
        grid=(N // BR,),
        in_specs=[
            pl.BlockSpec((BR, 1), lambda i: (i, 0)),
            pl.BlockSpec((BR, 1), lambda i: (i, 0)),
            pl.BlockSpec((BR, F_IN), lambda i: (i, 0)),
            pl.BlockSpec((F_IN, F), lambda i: (0, 0)),
        ],
        out_specs=[
            pl.BlockSpec((BR, F), lambda i: (i, 0)),
            pl.BlockSpec((BR, 1), lambda i: (i, 0)),
        ],
        out_shape=[
            jax.ShapeDtypeStruct((N, F), jnp.float32),
            jax.ShapeDtypeStruct((N, 1), jnp.float32),
        ],
    )(d0, d1, x, w)


def _tc_mid_body(p0_ref, p1_ref, g_ref, dism_ref, b_ref, w_ref, gout_ref):
    dism = dism_ref[...]
    agg = p0_ref[...] + p1_ref[...] + g_ref[...]
    h = jnp.maximum(agg * dism + b_ref[...], 0.0)
    gout_ref[...] = jnp.dot(h, w_ref[...],
                            preferred_element_type=jnp.float32) * dism


def _tc_mid(p0, p1, g, dism, b, w):
    return pl.pallas_call(
        _tc_mid_body,
        grid=(N // BR,),
        in_specs=[
            pl.BlockSpec((BR, F), lambda i: (i, 0)),
            pl.BlockSpec((BR, F), lambda i: (i, 0)),
            pl.BlockSpec((BR, F), lambda i: (i, 0)),
            pl.BlockSpec((BR, 1), lambda i: (i, 0)),
            pl.BlockSpec((1, F), lambda i: (0, 0)),
            pl.BlockSpec((F, F), lambda i: (0, 0)),
        ],
        out_specs=pl.BlockSpec((BR, F), lambda i: (i, 0)),
        out_shape=jax.ShapeDtypeStruct((N, F), jnp.float32),
    )(p0, p1, g, dism, b, w)


def _tc_final_body(p0_ref, p1_ref, g_ref, dism_ref, b_ref, out_ref):
    agg = p0_ref[...] + p1_ref[...] + g_ref[...]
    out_ref[...] = agg * dism_ref[...] + b_ref[...]


def _tc_final(p0, p1, g, dism, b):
    return pl.pallas_call(
        _tc_final_body,
        grid=(N // BR,),
        in_specs=[
            pl.BlockSpec((BR, F), lambda i: (i, 0)),
            pl.BlockSpec((BR, F), lambda i: (i, 0)),
            pl.BlockSpec((BR, F), lambda i: (i, 0)),
            pl.BlockSpec((BR, 1), lambda i: (i, 0)),
            pl.BlockSpec((1, F), lambda i: (0, 0)),
        ],
        out_specs=pl.BlockSpec((BR, F), lambda i: (i, 0)),
        out_shape=jax.ShapeDtypeStruct((N, F), jnp.float32),
    )(p0, p1, g, dism, b)


def kernel(x, edge_index, W0, b0, W1, b1, W2, b2):
    src = edge_index[0].astype(jnp.int32)
    dst = edge_index[1].astype(jnp.int32)
    src3 = jnp.concatenate(
        [src, jnp.zeros((EPAD - E,), jnp.int32)]).reshape(NC, NS, C, CHUNK)
    dst3 = jnp.concatenate(
        [dst, jnp.full((EPAD - E,), N, jnp.int32)]).reshape(NC, NS, C, CHUNK)

    ones_chunk = jnp.ones((CHUNK,), jnp.float32)
    zeros_deg = jnp.zeros((RW,), jnp.float32)
    zeros_rows = jnp.zeros((RWO, F), jnp.float32)

    degp = _deg_kernel(dst3, ones_chunk, zeros_deg)
    d0 = degp[0, :N].reshape(N, 1)
    d1 = degp[1, :N].reshape(N, 1)

    b0r = b0.reshape(1, F)
    b1r = b1.reshape(1, F)
    b2r = b2.reshape(1, F)

    g0, dism = _tc_first(d0, d1, x, W0)
    p = _agg_kernel(g0, src3, dst3, zeros_rows)
    g1 = _tc_mid(p[0], p[1], g0, dism, b0r, W1)
    p = _agg_kernel(g1, src3, dst3, zeros_rows)
    g2 = _tc_mid(p[0], p[1], g1, dism, b1r, W2)
    p = _agg_kernel(g2, src3, dst3, zeros_rows)
    return _tc_final(p[0], p[1], g2, dism, b2r)
